# Initial kernel scaffold; baseline (speedup 1.0000x reference)
import jax
import jax.numpy as jnp
from jax.experimental import pallas as pl


def kernel(x, edge_index, edge_attr, batch, W1, b1, W2, b2, W3, b3, root, b_conv, Wg, att_src, att_dst, b_gat, Wf1, bf1, Wf2, bf2, Wf3, bf3):
    raise NotImplementedError("write your pallas kernel here")


# speedup vs baseline: 2.6594x; 2.6594x over previous
import functools

import jax
import jax.numpy as jnp
from jax import lax
from jax.experimental import pallas as pl
from jax.experimental.pallas import tpu as pltpu
from jax.experimental.pallas import tpu_sc as plsc

N = 10000
E = 160000
D_IN = 128
G = 64
CHUNK = 128
NCHUNKS = 1280
E_PAD = NCHUNKS * CHUNK
NW = 32
CPW = NCHUNKS // NW
N_PAD = 10112
ROWS_PER_SUB = N_PAD // 16
BE = 2048
HIGH = lax.Precision.HIGHEST

_mesh = plsc.VectorSubcoreMesh(core_axis_name="c", subcore_axis_name="s")


@jax.jit
def _sc_gather(x, src_p):
    @functools.partial(
        pl.kernel,
        out_type=jax.ShapeDtypeStruct((E_PAD, D_IN), jnp.float32),
        mesh=_mesh,
        scratch_types=[
            pltpu.VMEM((CHUNK,), jnp.int32),
            pltpu.VMEM((CHUNK, D_IN), jnp.float32),
            pltpu.SemaphoreType.DMA,
        ],
    )
    def k(x_hbm, src_hbm, out_hbm, idx_v, rows_v, sem):
        c = lax.axis_index("c")
        s = lax.axis_index("s")
        w = s * 2 + c

        @pl.loop(0, CPW)
        def _(i):
            base = (i * NW + w) * CHUNK
            pltpu.sync_copy(src_hbm.at[pl.ds(base, CHUNK)], idx_v)
            pltpu.async_copy(x_hbm.at[idx_v], rows_v, sem).wait()
            pltpu.sync_copy(rows_v, out_hbm.at[pl.ds(base, CHUNK)])

    return k(x, src_p)


def _edge_body(ea_ref, xj_ref, w1_ref, b1_ref, w2_ref, b2_ref, w3_ref, out_ref):
    i = pl.program_id(0)
    h = jnp.maximum(jnp.dot(ea_ref[...], w1_ref[...], precision=HIGH) + b1_ref[...], 0.0)
    h = jnp.maximum(jnp.dot(h, w2_ref[...], precision=HIGH) + b2_ref[...], 0.0)
    v2 = jnp.dot(xj_ref[...], w3_ref[...], precision=HIGH)
    cols = []
    for o in range(8):
        mo = jnp.sum(h * v2[:, o * 64:(o + 1) * 64], axis=1, keepdims=True)
        cols.append(mo + v2[:, 512 + o:513 + o])
    cols.append(jnp.zeros((BE, 120), jnp.float32))
    msg = jnp.concatenate(cols, axis=1)
    rid = i * BE + lax.broadcasted_iota(jnp.int32, (BE, 1), 0)
    out_ref[...] = jnp.where(rid < E, msg, 0.0)


@jax.jit
def _tc_edge(ea_p, xj, W1, b1, W2, b2, W3big):
    return pl.pallas_call(
        _edge_body,
        grid=(E_PAD // BE,),
        in_specs=[
            pl.BlockSpec((BE, 16), lambda i: (i, 0)),
            pl.BlockSpec((BE, D_IN), lambda i: (i, 0)),
            pl.BlockSpec((16, 128), lambda i: (0, 0)),
            pl.BlockSpec((1, 128), lambda i: (0, 0)),
            pl.BlockSpec((128, 64), lambda i: (0, 0)),
            pl.BlockSpec((1, 64), lambda i: (0, 0)),
            pl.BlockSpec((128, 528), lambda i: (0, 0)),
        ],
        out_specs=pl.BlockSpec((BE, 128), lambda i: (i, 0)),
        out_shape=jax.ShapeDtypeStruct((E_PAD, 128), jnp.float32),
    )(ea_p, xj, W1, b1, W2, b2, W3big)


@jax.jit
def _sc_scatter(msg, dst_p):
    @functools.partial(
        pl.kernel,
        out_type=jax.ShapeDtypeStruct((2 * N_PAD, 128), jnp.float32),
        mesh=_mesh,
        scratch_types=[
            pltpu.VMEM((CHUNK,), jnp.int32),
            pltpu.VMEM((CHUNK, 128), jnp.float32),
            pltpu.VMEM_SHARED((N_PAD, 128), jnp.float32),
        ],
        compiler_params=pltpu.CompilerParams(needs_layout_passes=False),
    )
    def k(msg_hbm, dst_hbm, out_hbm, idx_v, rows_v, acc):
        c = lax.axis_index("c")
        s = lax.axis_index("s")

        @pl.loop(0, CHUNK)
        def _(j):
            for kk in range(8):
                rows_v[j, pl.ds(kk * 16, 16)] = jnp.zeros((16,), jnp.float32)

        for j in range(4):
            pltpu.sync_copy(rows_v,
                            acc.at[pl.ds(s * ROWS_PER_SUB + j * CHUNK, CHUNK)])
        pltpu.sync_copy(rows_v.at[pl.ds(0, ROWS_PER_SUB - 4 * CHUNK)],
                        acc.at[pl.ds(s * ROWS_PER_SUB + 4 * CHUNK,
                                     ROWS_PER_SUB - 4 * CHUNK)])
        plsc.subcore_barrier()

        @pl.loop(0, CPW)
        def _(i):
            base = (c * (NCHUNKS // 2) + i * 16 + s) * CHUNK
            pltpu.sync_copy(dst_hbm.at[pl.ds(base, CHUNK)], idx_v)
            pltpu.sync_copy(msg_hbm.at[pl.ds(base, CHUNK)], rows_v)
            pltpu.sync_copy(rows_v, acc.at[idx_v], add=True)

        plsc.subcore_barrier()
        pltpu.sync_copy(acc.at[pl.ds(s * ROWS_PER_SUB, ROWS_PER_SUB)],
                        out_hbm.at[pl.ds(c * N_PAD + s * ROWS_PER_SUB, ROWS_PER_SUB)])

    return k(msg, dst_p)


def _node_body(aggp_ref, x_ref, root_ref, bc_ref, wg_ref, as_ref, ad_ref,
               t_ref):
    agg = aggp_ref[0, 0:N, 0:8] + aggp_ref[1, 0:N, 0:8]
    xr = jnp.dot(x_ref[...], root_ref[...], precision=HIGH)
    h1 = jnp.maximum(agg + xr + bc_ref[...], 0.0)
    hw = jnp.dot(h1, wg_ref[...], precision=HIGH)
    a_s = jnp.sum(hw * as_ref[...], axis=1, keepdims=True)
    a_d = jnp.sum(hw * ad_ref[...], axis=1, keepdims=True)
    amax = jnp.max(a_s)
    zm = a_d + amax
    m = jnp.where(zm > 0, zm, 0.2 * zm)
    t_ref[...] = jnp.concatenate(
        [hw, a_s, a_d, m, jnp.zeros((N, 61), jnp.float32)], axis=1)


@jax.jit
def _tc_node(aggp, x, root, b_conv, Wg, att_src, att_dst):
    return pl.pallas_call(
        _node_body,
        out_shape=jax.ShapeDtypeStruct((N, 128), jnp.float32),
    )(aggp.reshape(2, N_PAD, 128), x, root, b_conv.reshape(1, 8),
      Wg, att_src.reshape(1, 64), att_dst.reshape(1, 64))


@jax.jit
def _sc_gat(T, src_p, dst_p):
    @functools.partial(
        pl.kernel,
        out_type=jax.ShapeDtypeStruct((2 * N_PAD, 128), jnp.float32),
        mesh=_mesh,
        scratch_types=[
            pltpu.VMEM((CHUNK,), jnp.int32),
            pltpu.VMEM((CHUNK,), jnp.int32),
            pltpu.VMEM((CHUNK, 128), jnp.float32),
            pltpu.VMEM((CHUNK, 128), jnp.float32),
            pltpu.VMEM((CHUNK, 128), jnp.float32),
            pltpu.VMEM_SHARED((N_PAD, 128), jnp.float32),
            pltpu.SemaphoreType.DMA,
        ],
        compiler_params=pltpu.CompilerParams(needs_layout_passes=False),
    )
    def k(t_hbm, src_hbm, dst_hbm, out_hbm,
          isv, idv, sbuf, dbuf, obuf, acc, sem):
        c = lax.axis_index("c")
        s = lax.axis_index("s")
        iota = lax.iota(jnp.int32, 16)

        @pl.loop(0, CHUNK)
        def _(j):
            for kk in range(8):
                obuf[j, pl.ds(kk * 16, 16)] = jnp.zeros((16,), jnp.float32)

        for j in range(4):
            pltpu.sync_copy(obuf,
                            acc.at[pl.ds(s * ROWS_PER_SUB + j * CHUNK, CHUNK)])
        pltpu.sync_copy(obuf.at[pl.ds(0, ROWS_PER_SUB - 4 * CHUNK)],
                        acc.at[pl.ds(s * ROWS_PER_SUB + 4 * CHUNK,
                                     ROWS_PER_SUB - 4 * CHUNK)])
        plsc.subcore_barrier()

        @pl.loop(0, CPW)
        def _(i):
            base = (c * (NCHUNKS // 2) + i * 16 + s) * CHUNK
            pltpu.sync_copy(src_hbm.at[pl.ds(base, CHUNK)], isv)
            pltpu.sync_copy(dst_hbm.at[pl.ds(base, CHUNK)], idv)
            pltpu.async_copy(t_hbm.at[isv], sbuf, sem).wait()
            pltpu.async_copy(t_hbm.at[idv], dbuf, sem).wait()

            @pl.loop(0, 8)
            def _(g):
                rowv = g * 16 + iota
                c64 = jnp.zeros((16,), jnp.int32) + 64
                a_sv = plsc.load_gather(sbuf, [rowv, c64])
                a_dv = plsc.load_gather(dbuf, [rowv, c64 + 1])
                mv = plsc.load_gather(dbuf, [rowv, c64 + 2])
                z = a_sv + a_dv
                ev = jnp.where(z > 0, z, 0.2 * z)
                exv = jnp.exp(ev - mv)
                gid = base + g * 16 + iota
                exv = jnp.where(gid < E, exv, 0.0)
                plsc.store_scatter(obuf, [rowv, c64], exv)

                @pl.loop(0, 64)
                def _(col):
                    cv = jnp.zeros((16,), jnp.int32) + col
                    hwc = plsc.load_gather(sbuf, [rowv, cv])
                    plsc.store_scatter(obuf, [rowv, cv], exv * hwc)

            pltpu.sync_copy(obuf, acc.at[idv], add=True)

        plsc.subcore_barrier()
        pltpu.sync_copy(acc.at[pl.ds(s * ROWS_PER_SUB, ROWS_PER_SUB)],
                        out_hbm.at[pl.ds(c * N_PAD + s * ROWS_PER_SUB, ROWS_PER_SUB)])

    return k(T, src_p, dst_p)


def _final_body(gatp_ref, t_ref, batch_ref, bg_ref,
                wf1_ref, bf1_ref, wf2_ref, bf2_ref, wf3_ref, bf3_ref, out_ref):
    hw = t_ref[:, 0:64]
    a_s = t_ref[:, 64:65]
    a_d = t_ref[:, 65:66]
    m = t_ref[:, 66:67]
    zs = a_s + a_d
    es = jnp.where(zs > 0, zs, 0.2 * zs)
    ds = jnp.exp(es - m)
    num = gatp_ref[0, 0:N, 0:64] + gatp_ref[1, 0:N, 0:64] + ds * hw
    den = gatp_ref[0, 0:N, 64:65] + gatp_ref[1, 0:N, 64:65] + ds
    h2 = jnp.maximum(num / (den + 1e-16) + bg_ref[...], 0.0)
    gi = lax.broadcasted_iota(jnp.int32, (G, N), 0)
    onehot = (gi == batch_ref[...]).astype(jnp.float32)
    sums = jnp.dot(onehot, h2, precision=HIGH)
    cnt = jnp.sum(onehot, axis=1, keepdims=True)
    pooled = sums / jnp.maximum(cnt, 1.0)
    o = jnp.maximum(jnp.dot(pooled, wf1_ref[...], precision=HIGH) + bf1_ref[...], 0.0)
    o = jnp.maximum(jnp.dot(o, wf2_ref[...], precision=HIGH) + bf2_ref[...], 0.0)
    o = jnp.maximum(jnp.dot(o, wf3_ref[...], precision=HIGH) + bf3_ref[...], 0.0)
    out_ref[...] = o


@jax.jit
def _tc_final(gatp, T, batch, b_gat, Wf1, bf1, Wf2, bf2, Wf3, bf3):
    return pl.pallas_call(
        _final_body,
        out_shape=jax.ShapeDtypeStruct((G, 32), jnp.float32),
    )(gatp.reshape(2, N_PAD, 128), T, batch.reshape(1, N), b_gat.reshape(1, 64),
      Wf1, bf1.reshape(1, 128), Wf2, bf2.reshape(1, 64), Wf3, bf3.reshape(1, 32))


def kernel(x, edge_index, edge_attr, batch, W1, b1, W2, b2, W3, b3, root,
           b_conv, Wg, att_src, att_dst, b_gat, Wf1, bf1, Wf2, bf2, Wf3, bf3):
    src = edge_index[0]
    dst = edge_index[1]
    pad = E_PAD - E
    src_p = jnp.concatenate([src, jnp.zeros((pad,), src.dtype)])
    dst_p = jnp.concatenate([dst, jnp.zeros((pad,), dst.dtype)])
    ea_p = jnp.concatenate([edge_attr, jnp.zeros((pad, 16), edge_attr.dtype)])

    W3r = W3.reshape(64, 128, 8)
    W3m2 = W3r.transpose(1, 2, 0).reshape(128, 512)
    b3r = b3.reshape(128, 8)
    W3big = jnp.concatenate([W3m2, b3r, jnp.zeros((128, 8), jnp.float32)], axis=1)

    xj = _sc_gather(x, src_p)
    msg = _tc_edge(ea_p, xj, W1, b1.reshape(1, 128), W2, b2.reshape(1, 64), W3big)
    aggp = _sc_scatter(msg, dst_p)
    T = _tc_node(aggp, x, root, b_conv, Wg, att_src, att_dst)
    gatp = _sc_gat(T, src_p, dst_p)
    return _tc_final(gatp, T, batch, b_gat, Wf1, bf1, Wf2, bf2, Wf3, bf3)


# speedup vs baseline: 2.9604x; 1.1132x over previous
import functools

import jax
import jax.numpy as jnp
from jax import lax
from jax.experimental import pallas as pl
from jax.experimental.pallas import tpu as pltpu
from jax.experimental.pallas import tpu_sc as plsc

N = 10000
E = 160000
D_IN = 128
G = 64
CHUNK = 128
NCHUNKS = 1280
E_PAD = NCHUNKS * CHUNK
NW = 32
CPW = NCHUNKS // NW
N_PAD = 10112
ROWS_PER_SUB = N_PAD // 16
BE = 2048
HIGH = lax.Precision.HIGHEST

_mesh = plsc.VectorSubcoreMesh(core_axis_name="c", subcore_axis_name="s")


@jax.jit
def _sc_gather(x, src_p):
    @functools.partial(
        pl.kernel,
        out_type=jax.ShapeDtypeStruct((E_PAD, D_IN), jnp.float32),
        mesh=_mesh,
        scratch_types=[
            pltpu.VMEM((CHUNK,), jnp.int32),
            pltpu.VMEM((CHUNK,), jnp.int32),
            pltpu.VMEM((CHUNK, D_IN), jnp.float32),
            pltpu.VMEM((CHUNK, D_IN), jnp.float32),
            pltpu.SemaphoreType.DMA,
            pltpu.SemaphoreType.DMA,
        ],
    )
    def k(x_hbm, src_hbm, out_hbm, iv0, iv1, rv0, rv1, sg0, sg1):
        c = lax.axis_index("c")
        s = lax.axis_index("s")
        w = s * 2 + c
        ivs, rvs, sgs = [iv0, iv1], [rv0, rv1], [sg0, sg1]

        def base(i):
            return (i * NW + w) * CHUNK

        def fetch_idx(i, b):
            pltpu.sync_copy(src_hbm.at[pl.ds(base(i), CHUNK)], ivs[b])

        def start_gather(b):
            pltpu.async_copy(x_hbm.at[ivs[b]], rvs[b], sgs[b])

        def wait_gather(b):
            pltpu.make_async_copy(x_hbm.at[ivs[b]], rvs[b], sgs[b]).wait()

        fetch_idx(0, 0)
        start_gather(0)

        @pl.loop(0, CPW, step=2)
        def _(i):
            for b in range(2):
                ch = i + b
                wait_gather(b)

                @pl.when(ch + 1 < CPW)
                def _():
                    fetch_idx(ch + 1, b ^ 1)
                    start_gather(b ^ 1)

                pltpu.sync_copy(rvs[b], out_hbm.at[pl.ds(base(ch), CHUNK)])

    return k(x, src_p)


def _edge_body(ea_ref, xj_ref, w1_ref, b1_ref, w2_ref, b2_ref, w3_ref, out_ref):
    i = pl.program_id(0)
    h = jnp.maximum(jnp.dot(ea_ref[...], w1_ref[...], precision=HIGH) + b1_ref[...], 0.0)
    h = jnp.maximum(jnp.dot(h, w2_ref[...], precision=HIGH) + b2_ref[...], 0.0)
    v2 = jnp.dot(xj_ref[...], w3_ref[...], precision=HIGH)
    cols = []
    for o in range(8):
        mo = jnp.sum(h * v2[:, o * 64:(o + 1) * 64], axis=1, keepdims=True)
        cols.append(mo + v2[:, 512 + o:513 + o])
    cols.append(jnp.zeros((BE, 120), jnp.float32))
    msg = jnp.concatenate(cols, axis=1)
    rid = i * BE + lax.broadcasted_iota(jnp.int32, (BE, 1), 0)
    out_ref[...] = jnp.where(rid < E, msg, 0.0)


@jax.jit
def _tc_edge(ea_p, xj, W1, b1, W2, b2, W3big):
    return pl.pallas_call(
        _edge_body,
        grid=(E_PAD // BE,),
        in_specs=[
            pl.BlockSpec((BE, 16), lambda i: (i, 0)),
            pl.BlockSpec((BE, D_IN), lambda i: (i, 0)),
            pl.BlockSpec((16, 128), lambda i: (0, 0)),
            pl.BlockSpec((1, 128), lambda i: (0, 0)),
            pl.BlockSpec((128, 64), lambda i: (0, 0)),
            pl.BlockSpec((1, 64), lambda i: (0, 0)),
            pl.BlockSpec((128, 528), lambda i: (0, 0)),
        ],
        out_specs=pl.BlockSpec((BE, 128), lambda i: (i, 0)),
        out_shape=jax.ShapeDtypeStruct((E_PAD, 128), jnp.float32),
    )(ea_p, xj, W1, b1, W2, b2, W3big)


@jax.jit
def _sc_scatter(msg, dst_p):
    @functools.partial(
        pl.kernel,
        out_type=jax.ShapeDtypeStruct((2 * N_PAD, 128), jnp.float32),
        mesh=_mesh,
        scratch_types=[
            pltpu.VMEM((CHUNK,), jnp.int32),
            pltpu.VMEM((CHUNK,), jnp.int32),
            pltpu.VMEM((CHUNK, 128), jnp.float32),
            pltpu.VMEM((CHUNK, 128), jnp.float32),
            pltpu.VMEM_SHARED((N_PAD, 128), jnp.float32),
            pltpu.SemaphoreType.DMA,
            pltpu.SemaphoreType.DMA,
        ],
        compiler_params=pltpu.CompilerParams(needs_layout_passes=False),
    )
    def k(msg_hbm, dst_hbm, out_hbm, iv0, iv1, rv0, rv1, acc, sm0, sm1):
        c = lax.axis_index("c")
        s = lax.axis_index("s")
        ivs, rvs, sms = [iv0, iv1], [rv0, rv1], [sm0, sm1]

        @pl.loop(0, CHUNK)
        def _(j):
            for kk in range(8):
                rv0[j, pl.ds(kk * 16, 16)] = jnp.zeros((16,), jnp.float32)

        for j in range(4):
            pltpu.sync_copy(rv0,
                            acc.at[pl.ds(s * ROWS_PER_SUB + j * CHUNK, CHUNK)])
        pltpu.sync_copy(rv0.at[pl.ds(0, ROWS_PER_SUB - 4 * CHUNK)],
                        acc.at[pl.ds(s * ROWS_PER_SUB + 4 * CHUNK,
                                     ROWS_PER_SUB - 4 * CHUNK)])
        plsc.subcore_barrier()

        def base(i):
            return (c * (NCHUNKS // 2) + i * 16 + s) * CHUNK

        def fetch(i, b):
            pltpu.sync_copy(dst_hbm.at[pl.ds(base(i), CHUNK)], ivs[b])
            pltpu.async_copy(msg_hbm.at[pl.ds(base(i), CHUNK)], rvs[b], sms[b])

        def wait_rows(i, b):
            pltpu.make_async_copy(msg_hbm.at[pl.ds(base(i), CHUNK)],
                                  rvs[b], sms[b]).wait()

        fetch(0, 0)

        @pl.loop(0, CPW, step=2)
        def _(i):
            for b in range(2):
                ch = i + b
                wait_rows(ch, b)

                @pl.when(ch + 1 < CPW)
                def _():
                    fetch(ch + 1, b ^ 1)

                pltpu.sync_copy(rvs[b], acc.at[ivs[b]], add=True)

        plsc.subcore_barrier()
        pltpu.sync_copy(acc.at[pl.ds(s * ROWS_PER_SUB, ROWS_PER_SUB)],
                        out_hbm.at[pl.ds(c * N_PAD + s * ROWS_PER_SUB, ROWS_PER_SUB)])

    return k(msg, dst_p)


def _node_body(aggp_ref, x_ref, root_ref, bc_ref, wg_ref, as_ref, ad_ref,
               t_ref):
    agg = aggp_ref[0, 0:N, 0:8] + aggp_ref[1, 0:N, 0:8]
    xr = jnp.dot(x_ref[...], root_ref[...], precision=HIGH)
    h1 = jnp.maximum(agg + xr + bc_ref[...], 0.0)
    hw = jnp.dot(h1, wg_ref[...], precision=HIGH)
    a_s = jnp.sum(hw * as_ref[...], axis=1, keepdims=True)
    a_d = jnp.sum(hw * ad_ref[...], axis=1, keepdims=True)
    zm = jnp.max(a_s) + jnp.max(a_d)
    ms = jnp.where(zm > 0, zm, 0.2 * zm)
    mcol = jnp.zeros((N, 1), jnp.float32) + ms
    t_ref[...] = jnp.concatenate(
        [hw, a_s, a_d, mcol, jnp.zeros((N, 61), jnp.float32)], axis=1)


@jax.jit
def _tc_node(aggp, x, root, b_conv, Wg, att_src, att_dst):
    return pl.pallas_call(
        _node_body,
        out_shape=jax.ShapeDtypeStruct((N, 128), jnp.float32),
    )(aggp.reshape(2, N_PAD, 128), x, root, b_conv.reshape(1, 8),
      Wg, att_src.reshape(1, 64), att_dst.reshape(1, 64))


@jax.jit
def _sc_gat(T, adm, idx2):
    @functools.partial(
        pl.kernel,
        out_type=jax.ShapeDtypeStruct((2 * N_PAD, 128), jnp.float32),
        mesh=_mesh,
        scratch_types=[
            pltpu.VMEM((2, CHUNK), jnp.int32),
            pltpu.VMEM((CHUNK, 128), jnp.float32),
            pltpu.VMEM((88, 128), jnp.float32),
            pltpu.VMEM((CHUNK, 128), jnp.float32),
            pltpu.VMEM_SHARED((N_PAD, 128), jnp.float32),
            pltpu.SemaphoreType.DMA,
        ],
        compiler_params=pltpu.CompilerParams(needs_layout_passes=False),
    )
    def k(t_hbm, adm_hbm, i2_hbm, out_hbm,
          ib, sbuf, adv, obuf, acc, sg):
        c = lax.axis_index("c")
        s = lax.axis_index("s")
        iota = lax.iota(jnp.int32, 16)
        pltpu.sync_copy(adm_hbm, adv)
        mvec = plsc.load_gather(
            adv, [jnp.zeros((16,), jnp.int32) + 80, jnp.zeros((16,), jnp.int32)])

        @pl.loop(0, CHUNK)
        def _(j):
            for kk in range(8):
                obuf[j, pl.ds(kk * 16, 16)] = jnp.zeros((16,), jnp.float32)

        for j in range(4):
            pltpu.sync_copy(obuf,
                            acc.at[pl.ds(s * ROWS_PER_SUB + j * CHUNK, CHUNK)])
        pltpu.sync_copy(obuf.at[pl.ds(0, ROWS_PER_SUB - 4 * CHUNK)],
                        acc.at[pl.ds(s * ROWS_PER_SUB + 4 * CHUNK,
                                     ROWS_PER_SUB - 4 * CHUNK)])
        plsc.subcore_barrier()

        def chunk(i):
            return c * (NCHUNKS // 2) + i * 16 + s

        def fetch_idx(i):
            pltpu.sync_copy(i2_hbm.at[pl.ds(2 * chunk(i), 2)], ib)

        def start_gather():
            pltpu.async_copy(t_hbm.at[ib.at[0]], sbuf, sg)

        def wait_gather():
            pltpu.make_async_copy(t_hbm.at[ib.at[0]], sbuf, sg).wait()

        @pl.loop(0, CPW)
        def _(i):
            fetch_idx(i)
            start_gather()
            wait_gather()
            base = chunk(i) * CHUNK

            @pl.loop(0, 8)
            def _(g):
                rowv = g * 16 + iota
                c64 = jnp.zeros((16,), jnp.int32) + 64
                a_sv = plsc.load_gather(sbuf, [rowv, c64])
                dstv = ib[1, pl.ds(g * 16, 16)]
                drow = lax.shift_right_logical(dstv, 7)
                dcol = lax.bitwise_and(dstv, 127)
                a_dv = plsc.load_gather(adv, [drow, dcol])
                z = a_sv + a_dv
                ev = jnp.where(z > 0, z, 0.2 * z)
                exv = jnp.exp(ev - mvec)
                gid = base + g * 16 + iota
                exv = jnp.where(gid < E, exv, 0.0)
                plsc.store_scatter(obuf, [rowv, c64], exv)

                for col in range(64):
                    cv = jnp.zeros((16,), jnp.int32) + col
                    hwc = plsc.load_gather(sbuf, [rowv, cv])
                    plsc.store_scatter(obuf, [rowv, cv], exv * hwc)

            pltpu.sync_copy(obuf, acc.at[ib.at[1]], add=True)

        plsc.subcore_barrier()
        pltpu.sync_copy(acc.at[pl.ds(s * ROWS_PER_SUB, ROWS_PER_SUB)],
                        out_hbm.at[pl.ds(c * N_PAD + s * ROWS_PER_SUB, ROWS_PER_SUB)])

    return k(T, adm, idx2)


def _final_body(gatp_ref, t_ref, batch_ref, bg_ref,
                wf1_ref, bf1_ref, wf2_ref, bf2_ref, wf3_ref, bf3_ref, out_ref):
    hw = t_ref[:, 0:64]
    a_s = t_ref[:, 64:65]
    a_d = t_ref[:, 65:66]
    m = t_ref[:, 66:67]
    zs = a_s + a_d
    es = jnp.where(zs > 0, zs, 0.2 * zs)
    ds = jnp.exp(es - m)
    num = gatp_ref[0, 0:N, 0:64] + gatp_ref[1, 0:N, 0:64] + ds * hw
    den = gatp_ref[0, 0:N, 64:65] + gatp_ref[1, 0:N, 64:65] + ds
    h2 = jnp.maximum(num / (den + 1e-16) + bg_ref[...], 0.0)
    gi = lax.broadcasted_iota(jnp.int32, (G, N), 0)
    onehot = (gi == batch_ref[...]).astype(jnp.float32)
    sums = jnp.dot(onehot, h2, precision=HIGH)
    cnt = jnp.sum(onehot, axis=1, keepdims=True)
    pooled = sums / jnp.maximum(cnt, 1.0)
    o = jnp.maximum(jnp.dot(pooled, wf1_ref[...], precision=HIGH) + bf1_ref[...], 0.0)
    o = jnp.maximum(jnp.dot(o, wf2_ref[...], precision=HIGH) + bf2_ref[...], 0.0)
    o = jnp.maximum(jnp.dot(o, wf3_ref[...], precision=HIGH) + bf3_ref[...], 0.0)
    out_ref[...] = o


@jax.jit
def _tc_final(gatp, T, batch, b_gat, Wf1, bf1, Wf2, bf2, Wf3, bf3):
    return pl.pallas_call(
        _final_body,
        out_shape=jax.ShapeDtypeStruct((G, 32), jnp.float32),
    )(gatp.reshape(2, N_PAD, 128), T, batch.reshape(1, N), b_gat.reshape(1, 64),
      Wf1, bf1.reshape(1, 128), Wf2, bf2.reshape(1, 64), Wf3, bf3.reshape(1, 32))


def kernel(x, edge_index, edge_attr, batch, W1, b1, W2, b2, W3, b3, root,
           b_conv, Wg, att_src, att_dst, b_gat, Wf1, bf1, Wf2, bf2, Wf3, bf3):
    src = edge_index[0]
    dst = edge_index[1]
    pad = E_PAD - E
    src_p = jnp.concatenate([src, jnp.zeros((pad,), src.dtype)])
    dst_p = jnp.concatenate([dst, jnp.zeros((pad,), dst.dtype)])
    ea_p = jnp.concatenate([edge_attr, jnp.zeros((pad, 16), edge_attr.dtype)])
    idx2 = jnp.stack([src_p.reshape(NCHUNKS, CHUNK),
                      dst_p.reshape(NCHUNKS, CHUNK)], axis=1).reshape(2 * NCHUNKS, CHUNK)

    W3r = W3.reshape(64, 128, 8)
    W3m2 = W3r.transpose(1, 2, 0).reshape(128, 512)
    b3r = b3.reshape(128, 8)
    W3big = jnp.concatenate([W3m2, b3r, jnp.zeros((128, 8), jnp.float32)], axis=1)

    xj = _sc_gather(x, src_p)
    msg = _tc_edge(ea_p, xj, W1, b1.reshape(1, 128), W2, b2.reshape(1, 64), W3big)
    aggp = _sc_scatter(msg, dst_p)
    T = _tc_node(aggp, x, root, b_conv, Wg, att_src, att_dst)
    adm = jnp.concatenate(
        [jnp.pad(T[:, 65], (0, 10240 - N)).reshape(80, 128),
         jnp.broadcast_to(T[0:1, 66:67], (8, 128))], axis=0)
    gatp = _sc_gat(T, adm, idx2)
    return _tc_final(gatp, T, batch, b_gat, Wf1, bf1, Wf2, bf2, Wf3, bf3)


# speedup vs baseline: 3.6232x; 1.2239x over previous
import functools

import jax
import jax.numpy as jnp
from jax import lax
from jax.experimental import pallas as pl
from jax.experimental.pallas import tpu as pltpu
from jax.experimental.pallas import tpu_sc as plsc

N = 10000
E = 160000
D_IN = 128
G = 64
CHUNK = 128
NCHUNKS = 1280
E_PAD = NCHUNKS * CHUNK
NW = 32
CPW = NCHUNKS // NW
N_PAD = 10112
ROWS_PER_SUB = N_PAD // 16
BE = 2048
HIGH = lax.Precision.HIGHEST

_mesh = plsc.VectorSubcoreMesh(core_axis_name="c", subcore_axis_name="s")


@jax.jit
def _sc_gather(x, src_p):
    @functools.partial(
        pl.kernel,
        out_type=jax.ShapeDtypeStruct((E_PAD, D_IN), jnp.float32),
        mesh=_mesh,
        scratch_types=[
            pltpu.VMEM((CHUNK,), jnp.int32),
            pltpu.VMEM((CHUNK,), jnp.int32),
            pltpu.VMEM((CHUNK, D_IN), jnp.float32),
            pltpu.VMEM((CHUNK, D_IN), jnp.float32),
            pltpu.SemaphoreType.DMA,
            pltpu.SemaphoreType.DMA,
            pltpu.SemaphoreType.DMA,
            pltpu.SemaphoreType.DMA,
        ],
    )
    def k(x_hbm, src_hbm, out_hbm, iv0, iv1, rv0, rv1, sg0, sg1, sw0, sw1):
        c = lax.axis_index("c")
        s = lax.axis_index("s")
        w = s * 2 + c
        ivs, rvs = [iv0, iv1], [rv0, rv1]
        sgs, sws = [sg0, sg1], [sw0, sw1]

        def base(i):
            return (i * NW + w) * CHUNK

        def fetch_idx(i, b):
            pltpu.sync_copy(src_hbm.at[pl.ds(base(i), CHUNK)], ivs[b])

        def start_gather(b):
            pltpu.async_copy(x_hbm.at[ivs[b]], rvs[b], sgs[b])

        def wait_gather(b):
            pltpu.make_async_copy(x_hbm.at[ivs[b]], rvs[b], sgs[b]).wait()

        def start_wb(i, b):
            pltpu.async_copy(rvs[b], out_hbm.at[pl.ds(base(i), CHUNK)], sws[b])

        def wait_wb(i, b):
            pltpu.make_async_copy(rvs[b], out_hbm.at[pl.ds(base(i), CHUNK)],
                                  sws[b]).wait()

        fetch_idx(0, 0)
        start_gather(0)

        @pl.loop(0, CPW, step=2)
        def _(i):
            for b in range(2):
                ch = i + b
                wait_gather(b)
                start_wb(ch, b)

                @pl.when(ch + 1 < CPW)
                def _():
                    fetch_idx(ch + 1, b ^ 1)

                    @pl.when(ch >= 1)
                    def _():
                        wait_wb(ch - 1, b ^ 1)

                    start_gather(b ^ 1)

        wait_wb(CPW - 2, 0)
        wait_wb(CPW - 1, 1)

    return k(x, src_p)


def _edge_body(ea_ref, xj_ref, w1_ref, b1_ref, w2_ref, b2_ref, w3_ref, out_ref):
    i = pl.program_id(0)
    h = jnp.maximum(jnp.dot(ea_ref[...], w1_ref[...], precision=HIGH) + b1_ref[...], 0.0)
    h = jnp.maximum(jnp.dot(h, w2_ref[...], precision=HIGH) + b2_ref[...], 0.0)
    v2 = jnp.dot(xj_ref[...], w3_ref[...], precision=HIGH)
    cols = []
    for o in range(8):
        mo = jnp.sum(h * v2[:, o * 64:(o + 1) * 64], axis=1, keepdims=True)
        cols.append(mo + v2[:, 512 + o:513 + o])
    cols.append(jnp.zeros((BE, 120), jnp.float32))
    msg = jnp.concatenate(cols, axis=1)
    rid = i * BE + lax.broadcasted_iota(jnp.int32, (BE, 1), 0)
    out_ref[...] = jnp.where(rid < E, msg, 0.0)


@jax.jit
def _tc_edge(ea_p, xj, W1, b1, W2, b2, W3big):
    return pl.pallas_call(
        _edge_body,
        grid=(E_PAD // BE,),
        in_specs=[
            pl.BlockSpec((BE, 16), lambda i: (i, 0)),
            pl.BlockSpec((BE, D_IN), lambda i: (i, 0)),
            pl.BlockSpec((16, 128), lambda i: (0, 0)),
            pl.BlockSpec((1, 128), lambda i: (0, 0)),
            pl.BlockSpec((128, 64), lambda i: (0, 0)),
            pl.BlockSpec((1, 64), lambda i: (0, 0)),
            pl.BlockSpec((128, 528), lambda i: (0, 0)),
        ],
        out_specs=pl.BlockSpec((BE, 128), lambda i: (i, 0)),
        out_shape=jax.ShapeDtypeStruct((E_PAD, 128), jnp.float32),
    )(ea_p, xj, W1, b1, W2, b2, W3big)


@jax.jit
def _sc_scatter(msg, dst_p):
    @functools.partial(
        pl.kernel,
        out_type=jax.ShapeDtypeStruct((2 * N_PAD, 128), jnp.float32),
        mesh=_mesh,
        scratch_types=[
            pltpu.VMEM((CHUNK,), jnp.int32),
            pltpu.VMEM((CHUNK,), jnp.int32),
            pltpu.VMEM((CHUNK, 128), jnp.float32),
            pltpu.VMEM((CHUNK, 128), jnp.float32),
            pltpu.VMEM_SHARED((N_PAD, 128), jnp.float32),
            pltpu.SemaphoreType.DMA,
            pltpu.SemaphoreType.DMA,
        ],
        compiler_params=pltpu.CompilerParams(needs_layout_passes=False),
    )
    def k(msg_hbm, dst_hbm, out_hbm, iv0, iv1, rv0, rv1, acc, sm0, sm1):
        c = lax.axis_index("c")
        s = lax.axis_index("s")
        ivs, rvs, sms = [iv0, iv1], [rv0, rv1], [sm0, sm1]

        @pl.loop(0, CHUNK)
        def _(j):
            for kk in range(8):
                rv0[j, pl.ds(kk * 16, 16)] = jnp.zeros((16,), jnp.float32)

        for j in range(4):
            pltpu.sync_copy(rv0,
                            acc.at[pl.ds(s * ROWS_PER_SUB + j * CHUNK, CHUNK)])
        pltpu.sync_copy(rv0.at[pl.ds(0, ROWS_PER_SUB - 4 * CHUNK)],
                        acc.at[pl.ds(s * ROWS_PER_SUB + 4 * CHUNK,
                                     ROWS_PER_SUB - 4 * CHUNK)])
        plsc.subcore_barrier()

        def base(i):
            return (c * (NCHUNKS // 2) + i * 16 + s) * CHUNK

        def fetch(i, b):
            pltpu.sync_copy(dst_hbm.at[pl.ds(base(i), CHUNK)], ivs[b])
            pltpu.async_copy(msg_hbm.at[pl.ds(base(i), CHUNK)], rvs[b], sms[b])

        def wait_rows(i, b):
            pltpu.make_async_copy(msg_hbm.at[pl.ds(base(i), CHUNK)],
                                  rvs[b], sms[b]).wait()

        fetch(0, 0)

        @pl.loop(0, CPW, step=2)
        def _(i):
            for b in range(2):
                ch = i + b
                wait_rows(ch, b)

                @pl.when(ch + 1 < CPW)
                def _():
                    fetch(ch + 1, b ^ 1)

                pltpu.sync_copy(rvs[b], acc.at[ivs[b]], add=True)

        plsc.subcore_barrier()
        pltpu.sync_copy(acc.at[pl.ds(s * ROWS_PER_SUB, ROWS_PER_SUB)],
                        out_hbm.at[pl.ds(c * N_PAD + s * ROWS_PER_SUB, ROWS_PER_SUB)])

    return k(msg, dst_p)


def _node_body(aggp_ref, x_ref, root_ref, bc_ref, wg_ref, as_ref, ad_ref,
               t_ref):
    agg = aggp_ref[0, 0:N, 0:8] + aggp_ref[1, 0:N, 0:8]
    xr = jnp.dot(x_ref[...], root_ref[...], precision=HIGH)
    h1 = jnp.maximum(agg + xr + bc_ref[...], 0.0)
    hw = jnp.dot(h1, wg_ref[...], precision=HIGH)
    a_s = jnp.sum(hw * as_ref[...], axis=1, keepdims=True)
    a_d = jnp.sum(hw * ad_ref[...], axis=1, keepdims=True)
    zm = jnp.max(a_s) + jnp.max(a_d)
    ms = jnp.where(zm > 0, zm, 0.2 * zm)
    mcol = jnp.zeros((N, 1), jnp.float32) + ms
    t_ref[...] = jnp.concatenate(
        [hw, a_s, a_d, mcol, jnp.zeros((N, 61), jnp.float32)], axis=1)


@jax.jit
def _tc_node(aggp, x, root, b_conv, Wg, att_src, att_dst):
    return pl.pallas_call(
        _node_body,
        out_shape=jax.ShapeDtypeStruct((N, 128), jnp.float32),
    )(aggp.reshape(2, N_PAD, 128), x, root, b_conv.reshape(1, 8),
      Wg, att_src.reshape(1, 64), att_dst.reshape(1, 64))


@jax.jit
def _sc_gat(T, adm, idx2):
    @functools.partial(
        pl.kernel,
        out_type=jax.ShapeDtypeStruct((2 * N_PAD, 128), jnp.float32),
        mesh=_mesh,
        scratch_types=[
            pltpu.VMEM((2, CHUNK), jnp.int32),
            pltpu.VMEM((2, CHUNK), jnp.int32),
            pltpu.VMEM((CHUNK, 128), jnp.float32),
            pltpu.VMEM((CHUNK, 128), jnp.float32),
            pltpu.VMEM((88, 128), jnp.float32),
            pltpu.VMEM((CHUNK,), jnp.float32),
            pltpu.VMEM_SHARED((N_PAD, 128), jnp.float32),
            pltpu.SemaphoreType.DMA,
            pltpu.SemaphoreType.DMA,
        ],
        compiler_params=pltpu.CompilerParams(needs_layout_passes=False),
    )
    def k(t_hbm, adm_hbm, i2_hbm, out_hbm,
          ib0, ib1, sb0, sb1, adv, exbuf, acc, sg0, sg1):
        c = lax.axis_index("c")
        s = lax.axis_index("s")
        iota = lax.iota(jnp.int32, 16)
        ibs, sbs, sgs = [ib0, ib1], [sb0, sb1], [sg0, sg1]
        pltpu.sync_copy(adm_hbm, adv)
        mvec = plsc.load_gather(
            adv, [jnp.zeros((16,), jnp.int32) + 80, jnp.zeros((16,), jnp.int32)])

        @pl.loop(0, CHUNK)
        def _(j):
            for kk in range(8):
                sb0[j, pl.ds(kk * 16, 16)] = jnp.zeros((16,), jnp.float32)

        for j in range(4):
            pltpu.sync_copy(sb0,
                            acc.at[pl.ds(s * ROWS_PER_SUB + j * CHUNK, CHUNK)])
        pltpu.sync_copy(sb0.at[pl.ds(0, ROWS_PER_SUB - 4 * CHUNK)],
                        acc.at[pl.ds(s * ROWS_PER_SUB + 4 * CHUNK,
                                     ROWS_PER_SUB - 4 * CHUNK)])
        plsc.subcore_barrier()

        def chunk(i):
            return c * (NCHUNKS // 2) + i * 16 + s

        def fetch_idx(i, b):
            pltpu.sync_copy(i2_hbm.at[pl.ds(2 * chunk(i), 2)], ibs[b])

        def start_gather(b):
            pltpu.async_copy(t_hbm.at[ibs[b].at[0]], sbs[b], sgs[b])

        def wait_gather(b):
            pltpu.make_async_copy(t_hbm.at[ibs[b].at[0]], sbs[b], sgs[b]).wait()

        fetch_idx(0, 0)
        start_gather(0)
        fetch_idx(1, 1)

        @pl.loop(0, CPW, step=2)
        def _(i):
            for b in range(2):
                ch = i + b
                wait_gather(b)

                @pl.when(ch + 1 < CPW)
                def _():
                    start_gather(b ^ 1)

                base = chunk(ch) * CHUNK

                @pl.loop(0, 8)
                def _(g):
                    rowv = g * 16 + iota
                    c64 = jnp.zeros((16,), jnp.int32) + 64
                    a_sv = plsc.load_gather(sbs[b], [rowv, c64])
                    dstv = ibs[b][1, pl.ds(g * 16, 16)]
                    drow = lax.shift_right_logical(dstv, 7)
                    dcol = lax.bitwise_and(dstv, 127)
                    a_dv = plsc.load_gather(adv, [drow, dcol])
                    z = a_sv + a_dv
                    ev = jnp.where(z > 0, z, 0.2 * z)
                    exv = jnp.exp(ev - mvec)
                    gid = base + g * 16 + iota
                    exv = jnp.where(gid < E, exv, 0.0)
                    exbuf[pl.ds(g * 16, 16)] = exv

                @pl.loop(0, CHUNK)
                def _(j):
                    eb = plsc.load_gather(exbuf, [jnp.zeros((16,), jnp.int32) + j])
                    for kk in range(4):
                        sl = pl.ds(kk * 16, 16)
                        sbs[b][j, sl] = eb * sbs[b][j, sl]
                    ebm = jnp.where(iota == 0, eb, 0.0)
                    sbs[b][j, pl.ds(64, 16)] = ebm

                pltpu.sync_copy(sbs[b], acc.at[ibs[b].at[1]], add=True)

                @pl.when(ch + 2 < CPW)
                def _():
                    fetch_idx(ch + 2, b)

        plsc.subcore_barrier()
        pltpu.sync_copy(acc.at[pl.ds(s * ROWS_PER_SUB, ROWS_PER_SUB)],
                        out_hbm.at[pl.ds(c * N_PAD + s * ROWS_PER_SUB, ROWS_PER_SUB)])

    return k(T, adm, idx2)


def _final_body(gatp_ref, t_ref, batch_ref, bg_ref,
                wf1_ref, bf1_ref, wf2_ref, bf2_ref, wf3_ref, bf3_ref, out_ref):
    hw = t_ref[:, 0:64]
    a_s = t_ref[:, 64:65]
    a_d = t_ref[:, 65:66]
    m = t_ref[:, 66:67]
    zs = a_s + a_d
    es = jnp.where(zs > 0, zs, 0.2 * zs)
    ds = jnp.exp(es - m)
    num = gatp_ref[0, 0:N, 0:64] + gatp_ref[1, 0:N, 0:64] + ds * hw
    den = gatp_ref[0, 0:N, 64:65] + gatp_ref[1, 0:N, 64:65] + ds
    h2 = jnp.maximum(num / (den + 1e-16) + bg_ref[...], 0.0)
    gi = lax.broadcasted_iota(jnp.int32, (G, N), 0)
    onehot = (gi == batch_ref[...]).astype(jnp.float32)
    sums = jnp.dot(onehot, h2, precision=HIGH)
    cnt = jnp.sum(onehot, axis=1, keepdims=True)
    pooled = sums / jnp.maximum(cnt, 1.0)
    o = jnp.maximum(jnp.dot(pooled, wf1_ref[...], precision=HIGH) + bf1_ref[...], 0.0)
    o = jnp.maximum(jnp.dot(o, wf2_ref[...], precision=HIGH) + bf2_ref[...], 0.0)
    o = jnp.maximum(jnp.dot(o, wf3_ref[...], precision=HIGH) + bf3_ref[...], 0.0)
    out_ref[...] = o


@jax.jit
def _tc_final(gatp, T, batch, b_gat, Wf1, bf1, Wf2, bf2, Wf3, bf3):
    return pl.pallas_call(
        _final_body,
        out_shape=jax.ShapeDtypeStruct((G, 32), jnp.float32),
    )(gatp.reshape(2, N_PAD, 128), T, batch.reshape(1, N), b_gat.reshape(1, 64),
      Wf1, bf1.reshape(1, 128), Wf2, bf2.reshape(1, 64), Wf3, bf3.reshape(1, 32))


def kernel(x, edge_index, edge_attr, batch, W1, b1, W2, b2, W3, b3, root,
           b_conv, Wg, att_src, att_dst, b_gat, Wf1, bf1, Wf2, bf2, Wf3, bf3):
    src = edge_index[0]
    dst = edge_index[1]
    pad = E_PAD - E
    src_p = jnp.concatenate([src, jnp.zeros((pad,), src.dtype)])
    dst_p = jnp.concatenate([dst, jnp.zeros((pad,), dst.dtype)])
    ea_p = jnp.concatenate([edge_attr, jnp.zeros((pad, 16), edge_attr.dtype)])
    idx2 = jnp.stack([src_p.reshape(NCHUNKS, CHUNK),
                      dst_p.reshape(NCHUNKS, CHUNK)], axis=1).reshape(2 * NCHUNKS, CHUNK)

    W3r = W3.reshape(64, 128, 8)
    W3m2 = W3r.transpose(1, 2, 0).reshape(128, 512)
    b3r = b3.reshape(128, 8)
    W3big = jnp.concatenate([W3m2, b3r, jnp.zeros((128, 8), jnp.float32)], axis=1)

    xj = _sc_gather(x, src_p)
    msg = _tc_edge(ea_p, xj, W1, b1.reshape(1, 128), W2, b2.reshape(1, 64), W3big)
    aggp = _sc_scatter(msg, dst_p)
    T = _tc_node(aggp, x, root, b_conv, Wg, att_src, att_dst)
    adm = jnp.concatenate(
        [jnp.pad(T[:, 65], (0, 10240 - N)).reshape(80, 128),
         jnp.broadcast_to(T[0:1, 66:67], (8, 128))], axis=0)
    gatp = _sc_gat(T, adm, idx2)
    return _tc_final(gatp, T, batch, b_gat, Wf1, bf1, Wf2, bf2, Wf3, bf3)


# speedup vs baseline: 3.6553x; 1.0089x over previous
import functools

import jax
import jax.numpy as jnp
from jax import lax
from jax.experimental import pallas as pl
from jax.experimental.pallas import tpu as pltpu
from jax.experimental.pallas import tpu_sc as plsc

N = 10000
E = 160000
D_IN = 128
G = 64
CHUNK = 128
NCHUNKS = 1280
E_PAD = NCHUNKS * CHUNK
NW = 32
CPW = NCHUNKS // NW
N_PAD = 10112
ROWS_PER_SUB = N_PAD // 16
BE = 2048
HIGH = lax.Precision.HIGHEST

_mesh = plsc.VectorSubcoreMesh(core_axis_name="c", subcore_axis_name="s")


@jax.jit
def _sc_gather(x, src_p):
    @functools.partial(
        pl.kernel,
        out_type=jax.ShapeDtypeStruct((E_PAD, D_IN), jnp.float32),
        mesh=_mesh,
        scratch_types=[
            pltpu.VMEM((CHUNK,), jnp.int32),
            pltpu.VMEM((CHUNK,), jnp.int32),
            pltpu.VMEM((CHUNK,), jnp.int32),
            pltpu.VMEM((CHUNK,), jnp.int32),
            pltpu.VMEM((CHUNK, D_IN), jnp.float32),
            pltpu.VMEM((CHUNK, D_IN), jnp.float32),
            pltpu.VMEM((CHUNK, D_IN), jnp.float32),
            pltpu.VMEM((CHUNK, D_IN), jnp.float32),
            pltpu.SemaphoreType.DMA,
            pltpu.SemaphoreType.DMA,
            pltpu.SemaphoreType.DMA,
            pltpu.SemaphoreType.DMA,
            pltpu.SemaphoreType.DMA,
            pltpu.SemaphoreType.DMA,
            pltpu.SemaphoreType.DMA,
            pltpu.SemaphoreType.DMA,
        ],
    )
    def k(x_hbm, src_hbm, out_hbm, iv0, iv1, iv2, iv3, rv0, rv1, rv2, rv3,
          sg0, sg1, sg2, sg3, sw0, sw1, sw2, sw3):
        c = lax.axis_index("c")
        s = lax.axis_index("s")
        w = s * 2 + c
        ivs, rvs = [iv0, iv1, iv2, iv3], [rv0, rv1, rv2, rv3]
        sgs, sws = [sg0, sg1, sg2, sg3], [sw0, sw1, sw2, sw3]

        def base(i):
            return (i * NW + w) * CHUNK

        def fetch_idx(i, b):
            pltpu.sync_copy(src_hbm.at[pl.ds(base(i), CHUNK)], ivs[b])

        def start_gather(b):
            pltpu.async_copy(x_hbm.at[ivs[b]], rvs[b], sgs[b])

        def wait_gather(b):
            pltpu.make_async_copy(x_hbm.at[ivs[b]], rvs[b], sgs[b]).wait()

        def start_wb(i, b):
            pltpu.async_copy(rvs[b], out_hbm.at[pl.ds(base(i), CHUNK)], sws[b])

        def wait_wb(i, b):
            pltpu.make_async_copy(rvs[b], out_hbm.at[pl.ds(base(i), CHUNK)],
                                  sws[b]).wait()

        for q in range(3):
            fetch_idx(q, q)
            start_gather(q)

        @pl.loop(0, CPW, step=4)
        def _(i):
            for b in range(4):
                ch = i + b
                b3 = (b + 3) % 4
                wait_gather(b)
                start_wb(ch, b)

                @pl.when(ch + 3 < CPW)
                def _():
                    fetch_idx(ch + 3, b3)

                    @pl.when(ch >= 1)
                    def _():
                        wait_wb(ch - 1, b3)

                    start_gather(b3)

        for q in range(4):
            wait_wb(CPW - 4 + q, q)

    return k(x, src_p)


def _edge_body(ea_ref, xj_ref, w1_ref, b1_ref, w2_ref, b2_ref, w3_ref, out_ref):
    i = pl.program_id(0)
    h = jnp.maximum(jnp.dot(ea_ref[...], w1_ref[...], precision=HIGH) + b1_ref[...], 0.0)
    h = jnp.maximum(jnp.dot(h, w2_ref[...], precision=HIGH) + b2_ref[...], 0.0)
    v2 = jnp.dot(xj_ref[...], w3_ref[...], precision=HIGH)
    cols = []
    for o in range(8):
        mo = jnp.sum(h * v2[:, o * 64:(o + 1) * 64], axis=1, keepdims=True)
        cols.append(mo + v2[:, 512 + o:513 + o])
    cols.append(jnp.zeros((BE, 120), jnp.float32))
    msg = jnp.concatenate(cols, axis=1)
    rid = i * BE + lax.broadcasted_iota(jnp.int32, (BE, 1), 0)
    out_ref[...] = jnp.where(rid < E, msg, 0.0)


@jax.jit
def _tc_edge(ea_p, xj, W1, b1, W2, b2, W3big):
    return pl.pallas_call(
        _edge_body,
        grid=(E_PAD // BE,),
        in_specs=[
            pl.BlockSpec((BE, 16), lambda i: (i, 0)),
            pl.BlockSpec((BE, D_IN), lambda i: (i, 0)),
            pl.BlockSpec((16, 128), lambda i: (0, 0)),
            pl.BlockSpec((1, 128), lambda i: (0, 0)),
            pl.BlockSpec((128, 64), lambda i: (0, 0)),
            pl.BlockSpec((1, 64), lambda i: (0, 0)),
            pl.BlockSpec((128, 528), lambda i: (0, 0)),
        ],
        out_specs=pl.BlockSpec((BE, 128), lambda i: (i, 0)),
        out_shape=jax.ShapeDtypeStruct((E_PAD, 128), jnp.float32),
    )(ea_p, xj, W1, b1, W2, b2, W3big)


@jax.jit
def _sc_scatter(msg, dst_p):
    @functools.partial(
        pl.kernel,
        out_type=jax.ShapeDtypeStruct((2 * N_PAD, 128), jnp.float32),
        mesh=_mesh,
        scratch_types=[
            pltpu.VMEM((CHUNK,), jnp.int32),
            pltpu.VMEM((CHUNK,), jnp.int32),
            pltpu.VMEM((CHUNK, 128), jnp.float32),
            pltpu.VMEM((CHUNK, 128), jnp.float32),
            pltpu.VMEM_SHARED((N_PAD, 128), jnp.float32),
            pltpu.SemaphoreType.DMA,
            pltpu.SemaphoreType.DMA,
        ],
        compiler_params=pltpu.CompilerParams(needs_layout_passes=False),
    )
    def k(msg_hbm, dst_hbm, out_hbm, iv0, iv1, rv0, rv1, acc, sm0, sm1):
        c = lax.axis_index("c")
        s = lax.axis_index("s")
        ivs, rvs, sms = [iv0, iv1], [rv0, rv1], [sm0, sm1]

        @pl.loop(0, CHUNK)
        def _(j):
            for kk in range(8):
                rv0[j, pl.ds(kk * 16, 16)] = jnp.zeros((16,), jnp.float32)

        for j in range(4):
            pltpu.sync_copy(rv0,
                            acc.at[pl.ds(s * ROWS_PER_SUB + j * CHUNK, CHUNK)])
        pltpu.sync_copy(rv0.at[pl.ds(0, ROWS_PER_SUB - 4 * CHUNK)],
                        acc.at[pl.ds(s * ROWS_PER_SUB + 4 * CHUNK,
                                     ROWS_PER_SUB - 4 * CHUNK)])
        plsc.subcore_barrier()

        def base(i):
            return (c * (NCHUNKS // 2) + i * 16 + s) * CHUNK

        def fetch(i, b):
            pltpu.sync_copy(dst_hbm.at[pl.ds(base(i), CHUNK)], ivs[b])
            pltpu.async_copy(msg_hbm.at[pl.ds(base(i), CHUNK)], rvs[b], sms[b])

        def wait_rows(i, b):
            pltpu.make_async_copy(msg_hbm.at[pl.ds(base(i), CHUNK)],
                                  rvs[b], sms[b]).wait()

        fetch(0, 0)

        @pl.loop(0, CPW, step=2)
        def _(i):
            for b in range(2):
                ch = i + b
                wait_rows(ch, b)

                @pl.when(ch + 1 < CPW)
                def _():
                    fetch(ch + 1, b ^ 1)

                pltpu.sync_copy(rvs[b], acc.at[ivs[b]], add=True)

        plsc.subcore_barrier()
        pltpu.sync_copy(acc.at[pl.ds(s * ROWS_PER_SUB, ROWS_PER_SUB)],
                        out_hbm.at[pl.ds(c * N_PAD + s * ROWS_PER_SUB, ROWS_PER_SUB)])

    return k(msg, dst_p)


def _node_body(aggp_ref, x_ref, root_ref, bc_ref, wg_ref, as_ref, ad_ref,
               t_ref):
    agg = aggp_ref[0, 0:N, 0:8] + aggp_ref[1, 0:N, 0:8]
    xr = jnp.dot(x_ref[...], root_ref[...], precision=HIGH)
    h1 = jnp.maximum(agg + xr + bc_ref[...], 0.0)
    hw = jnp.dot(h1, wg_ref[...], precision=HIGH)
    a_s = jnp.sum(hw * as_ref[...], axis=1, keepdims=True)
    a_d = jnp.sum(hw * ad_ref[...], axis=1, keepdims=True)
    zm = jnp.max(a_s) + jnp.max(a_d)
    ms = jnp.where(zm > 0, zm, 0.2 * zm)
    mcol = jnp.zeros((N, 1), jnp.float32) + ms
    t_ref[...] = jnp.concatenate(
        [hw, a_s, a_d, mcol, jnp.zeros((N, 61), jnp.float32)], axis=1)


@jax.jit
def _tc_node(aggp, x, root, b_conv, Wg, att_src, att_dst):
    return pl.pallas_call(
        _node_body,
        out_shape=jax.ShapeDtypeStruct((N, 128), jnp.float32),
    )(aggp.reshape(2, N_PAD, 128), x, root, b_conv.reshape(1, 8),
      Wg, att_src.reshape(1, 64), att_dst.reshape(1, 64))


@jax.jit
def _sc_gat(T, adm, idx2):
    @functools.partial(
        pl.kernel,
        out_type=jax.ShapeDtypeStruct((2 * N_PAD, 128), jnp.float32),
        mesh=_mesh,
        scratch_types=[
            pltpu.VMEM((2, CHUNK), jnp.int32),
            pltpu.VMEM((2, CHUNK), jnp.int32),
            pltpu.VMEM((CHUNK, 128), jnp.float32),
            pltpu.VMEM((CHUNK, 128), jnp.float32),
            pltpu.VMEM((88, 128), jnp.float32),
            pltpu.VMEM((CHUNK,), jnp.float32),
            pltpu.VMEM_SHARED((N_PAD, 128), jnp.float32),
            pltpu.SemaphoreType.DMA,
            pltpu.SemaphoreType.DMA,
        ],
        compiler_params=pltpu.CompilerParams(needs_layout_passes=False),
    )
    def k(t_hbm, adm_hbm, i2_hbm, out_hbm,
          ib0, ib1, sb0, sb1, adv, exbuf, acc, sg0, sg1):
        c = lax.axis_index("c")
        s = lax.axis_index("s")
        iota = lax.iota(jnp.int32, 16)
        ibs, sbs, sgs = [ib0, ib1], [sb0, sb1], [sg0, sg1]
        pltpu.sync_copy(adm_hbm, adv)
        mvec = plsc.load_gather(
            adv, [jnp.zeros((16,), jnp.int32) + 80, jnp.zeros((16,), jnp.int32)])

        @pl.loop(0, CHUNK)
        def _(j):
            for kk in range(8):
                sb0[j, pl.ds(kk * 16, 16)] = jnp.zeros((16,), jnp.float32)

        for j in range(4):
            pltpu.sync_copy(sb0,
                            acc.at[pl.ds(s * ROWS_PER_SUB + j * CHUNK, CHUNK)])
        pltpu.sync_copy(sb0.at[pl.ds(0, ROWS_PER_SUB - 4 * CHUNK)],
                        acc.at[pl.ds(s * ROWS_PER_SUB + 4 * CHUNK,
                                     ROWS_PER_SUB - 4 * CHUNK)])
        plsc.subcore_barrier()

        def chunk(i):
            return c * (NCHUNKS // 2) + i * 16 + s

        def fetch_idx(i, b):
            pltpu.sync_copy(i2_hbm.at[pl.ds(2 * chunk(i), 2)], ibs[b])

        def start_gather(b):
            pltpu.async_copy(t_hbm.at[ibs[b].at[0]], sbs[b], sgs[b])

        def wait_gather(b):
            pltpu.make_async_copy(t_hbm.at[ibs[b].at[0]], sbs[b], sgs[b]).wait()

        fetch_idx(0, 0)
        start_gather(0)
        fetch_idx(1, 1)

        @pl.loop(0, CPW, step=2)
        def _(i):
            for b in range(2):
                ch = i + b
                wait_gather(b)

                @pl.when(ch + 1 < CPW)
                def _():
                    start_gather(b ^ 1)

                base = chunk(ch) * CHUNK

                @pl.loop(0, 8)
                def _(g):
                    rowv = g * 16 + iota
                    c64 = jnp.zeros((16,), jnp.int32) + 64
                    a_sv = plsc.load_gather(sbs[b], [rowv, c64])
                    dstv = ibs[b][1, pl.ds(g * 16, 16)]
                    drow = lax.shift_right_logical(dstv, 7)
                    dcol = lax.bitwise_and(dstv, 127)
                    a_dv = plsc.load_gather(adv, [drow, dcol])
                    z = a_sv + a_dv
                    ev = jnp.where(z > 0, z, 0.2 * z)
                    exv = jnp.exp(ev - mvec)
                    gid = base + g * 16 + iota
                    exv = jnp.where(gid < E, exv, 0.0)
                    exbuf[pl.ds(g * 16, 16)] = exv

                @pl.loop(0, CHUNK)
                def _(j):
                    eb = plsc.load_gather(exbuf, [jnp.zeros((16,), jnp.int32) + j])
                    for kk in range(4):
                        sl = pl.ds(kk * 16, 16)
                        sbs[b][j, sl] = eb * sbs[b][j, sl]
                    ebm = jnp.where(iota == 0, eb, 0.0)
                    sbs[b][j, pl.ds(64, 16)] = ebm

                pltpu.sync_copy(sbs[b], acc.at[ibs[b].at[1]], add=True)

                @pl.when(ch + 2 < CPW)
                def _():
                    fetch_idx(ch + 2, b)

        plsc.subcore_barrier()
        pltpu.sync_copy(acc.at[pl.ds(s * ROWS_PER_SUB, ROWS_PER_SUB)],
                        out_hbm.at[pl.ds(c * N_PAD + s * ROWS_PER_SUB, ROWS_PER_SUB)])

    return k(T, adm, idx2)


def _final_body(gatp_ref, t_ref, batch_ref, bg_ref,
                wf1_ref, bf1_ref, wf2_ref, bf2_ref, wf3_ref, bf3_ref, out_ref):
    hw = t_ref[:, 0:64]
    a_s = t_ref[:, 64:65]
    a_d = t_ref[:, 65:66]
    m = t_ref[:, 66:67]
    zs = a_s + a_d
    es = jnp.where(zs > 0, zs, 0.2 * zs)
    ds = jnp.exp(es - m)
    num = gatp_ref[0, 0:N, 0:64] + gatp_ref[1, 0:N, 0:64] + ds * hw
    den = gatp_ref[0, 0:N, 64:65] + gatp_ref[1, 0:N, 64:65] + ds
    h2 = jnp.maximum(num / (den + 1e-16) + bg_ref[...], 0.0)
    gi = lax.broadcasted_iota(jnp.int32, (G, N), 0)
    onehot = (gi == batch_ref[...]).astype(jnp.float32)
    sums = jnp.dot(onehot, h2, precision=HIGH)
    cnt = jnp.sum(onehot, axis=1, keepdims=True)
    pooled = sums / jnp.maximum(cnt, 1.0)
    o = jnp.maximum(jnp.dot(pooled, wf1_ref[...], precision=HIGH) + bf1_ref[...], 0.0)
    o = jnp.maximum(jnp.dot(o, wf2_ref[...], precision=HIGH) + bf2_ref[...], 0.0)
    o = jnp.maximum(jnp.dot(o, wf3_ref[...], precision=HIGH) + bf3_ref[...], 0.0)
    out_ref[...] = o


@jax.jit
def _tc_final(gatp, T, batch, b_gat, Wf1, bf1, Wf2, bf2, Wf3, bf3):
    return pl.pallas_call(
        _final_body,
        out_shape=jax.ShapeDtypeStruct((G, 32), jnp.float32),
    )(gatp.reshape(2, N_PAD, 128), T, batch.reshape(1, N), b_gat.reshape(1, 64),
      Wf1, bf1.reshape(1, 128), Wf2, bf2.reshape(1, 64), Wf3, bf3.reshape(1, 32))


def kernel(x, edge_index, edge_attr, batch, W1, b1, W2, b2, W3, b3, root,
           b_conv, Wg, att_src, att_dst, b_gat, Wf1, bf1, Wf2, bf2, Wf3, bf3):
    src = edge_index[0]
    dst = edge_index[1]
    pad = E_PAD - E
    src_p = jnp.concatenate([src, jnp.zeros((pad,), src.dtype)])
    dst_p = jnp.concatenate([dst, jnp.zeros((pad,), dst.dtype)])
    ea_p = jnp.concatenate([edge_attr, jnp.zeros((pad, 16), edge_attr.dtype)])
    idx2 = jnp.stack([src_p.reshape(NCHUNKS, CHUNK),
                      dst_p.reshape(NCHUNKS, CHUNK)], axis=1).reshape(2 * NCHUNKS, CHUNK)

    W3r = W3.reshape(64, 128, 8)
    W3m2 = W3r.transpose(1, 2, 0).reshape(128, 512)
    b3r = b3.reshape(128, 8)
    W3big = jnp.concatenate([W3m2, b3r, jnp.zeros((128, 8), jnp.float32)], axis=1)

    xj = _sc_gather(x, src_p)
    msg = _tc_edge(ea_p, xj, W1, b1.reshape(1, 128), W2, b2.reshape(1, 64), W3big)
    aggp = _sc_scatter(msg, dst_p)
    T = _tc_node(aggp, x, root, b_conv, Wg, att_src, att_dst)
    adm = jnp.concatenate(
        [jnp.pad(T[:, 65], (0, 10240 - N)).reshape(80, 128),
         jnp.broadcast_to(T[0:1, 66:67], (8, 128))], axis=0)
    gatp = _sc_gat(T, adm, idx2)
    return _tc_final(gatp, T, batch, b_gat, Wf1, bf1, Wf2, bf2, Wf3, bf3)
